# Initial kernel scaffold; baseline (speedup 1.0000x reference)
import jax
import jax.numpy as jnp
from jax.experimental import pallas as pl


def kernel(x, edge_index, edge_attr, node_tables, edge_tables):
    raise NotImplementedError("write your pallas kernel here")


# speedup vs baseline: 12.6829x; 12.6829x over previous
import functools

import jax
import jax.numpy as jnp
from jax import lax
from jax.experimental import pallas as pl
from jax.experimental.pallas import tpu as pltpu
from jax.experimental.pallas import tpu_sc as plsc

_NF = 8
_SUB = 16
_C = 256
_G = _C // 128


@functools.lru_cache(maxsize=None)
def _build(N, E, node_k, edge_k):
    info = plsc.get_sparse_core_info()
    NC, NS = info.num_cores, info.num_subcores
    NW = NC * NS

    mesh = plsc.VectorSubcoreMesh(core_axis_name="c", subcore_axis_name="s")

    @functools.partial(
        pl.kernel,
        mesh=mesh,
        compiler_params=pltpu.CompilerParams(use_tc_tiling_on_sc=False),
        out_type=(
            jax.ShapeDtypeStruct((N, _NF * _SUB), jnp.float32),
            jax.ShapeDtypeStruct((E, _NF * _SUB), jnp.float32),
        ),
        scratch_types=[
            pltpu.VMEM((_NF, _C), jnp.float32),
            pltpu.VMEM((_NF, _G, 128), jnp.int32),
            pltpu.VMEM((_NF, _C, _SUB), jnp.float32),
            pltpu.SemaphoreType.DMA,
        ],
    )
    def tok(xt, at, ntab, etab, out_x, out_e, cols, idx, rows, sem):
        wid = lax.axis_index("s") * NC + lax.axis_index("c")

        def process(src_t, tab, out, total, K):
            n_chunks = (total + _C - 1) // _C
            my_n = (n_chunks - wid + NW - 1) // NW

            def chunk(t, carry):
                cid = wid + t * NW
                base = jnp.minimum(cid * _C, total - _C)
                pltpu.sync_copy(src_t.at[:, pl.ds(base, _C)], cols)
                copies = []
                for i in range(_NF):
                    for g in range(_G):
                        for j in range(8):
                            off = g * 128 + j * 16
                            col = cols[i, pl.ds(off, 16)]
                            v = (col * float(K)).astype(jnp.int32) + 1
                            v = jnp.where(col == -1.0, 0, v)
                            v = jnp.clip(v, 0, K - 1) + i * K
                            idx[i, g, pl.ds(j * 16, 16)] = v
                        copies.append(pltpu.async_copy(
                            tab.at[idx.at[i, g]],
                            rows.at[i, pl.ds(g * 128, 128)],
                            sem,
                        ))
                for cp in copies:
                    cp.wait()
                for i in range(_NF):
                    pltpu.sync_copy(
                        rows.at[i],
                        out.at[pl.ds(base, _C), pl.ds(i * _SUB, _SUB)],
                    )
                return carry

            lax.fori_loop(0, my_n, chunk, 0)

        process(xt, ntab, out_x, N, node_k)
        process(at, etab, out_e, E, edge_k)

    return tok


@jax.jit
def kernel(x, edge_index, edge_attr, node_tables, edge_tables):
    del edge_index
    N = x.shape[0]
    E = edge_attr.shape[0]
    node_k = node_tables.shape[1]
    edge_k = edge_tables.shape[1]
    tok = _build(N, E, node_k, edge_k)
    out_x, out_e = tok(
        x.T,
        edge_attr.T,
        node_tables.reshape(-1, _SUB),
        edge_tables.reshape(-1, _SUB),
    )
    return (out_x, out_e)


# speedup vs baseline: 12.7469x; 1.0050x over previous
import functools

import jax
import jax.numpy as jnp
from jax import lax
from jax.experimental import pallas as pl
from jax.experimental.pallas import tpu as pltpu
from jax.experimental.pallas import tpu_sc as plsc

_NF = 8
_SUB = 16
_C = 256
_V = _C * _NF
_G = _V // 128


@functools.lru_cache(maxsize=None)
def _build(N, E, node_k, edge_k):
    info = plsc.get_sparse_core_info()
    NC, NS = info.num_cores, info.num_subcores
    NW = NC * NS
    mesh = plsc.VectorSubcoreMesh(core_axis_name="c", subcore_axis_name="s")

    @functools.partial(
        pl.kernel,
        mesh=mesh,
        compiler_params=pltpu.CompilerParams(use_tc_tiling_on_sc=False),
        out_type=(
            jax.ShapeDtypeStruct((N * _NF, _SUB), jnp.float32),
            jax.ShapeDtypeStruct((E * _NF, _SUB), jnp.float32),
        ),
        scratch_types=[
            pltpu.VMEM((2, _V), jnp.float32),
            pltpu.VMEM((2, _G, 128), jnp.int32),
            pltpu.VMEM((2, _V, _SUB), jnp.float32),
            pltpu.SemaphoreType.DMA,
            pltpu.SemaphoreType.DMA,
            pltpu.SemaphoreType.DMA,
            pltpu.SemaphoreType.DMA,
            pltpu.SemaphoreType.DMA,
        ],
    )
    def tok(xa, ea, ntab, etab, out_x, out_e,
            attr, idx, rows, sema0, sema1, semg, semw0, semw1):
        wid = lax.axis_index("s") * NC + lax.axis_index("c")
        iota = lax.broadcasted_iota(jnp.int32, (16,), 0)
        field = jnp.bitwise_and(iota, _NF - 1)
        sema = (sema0, sema1)
        semw = (semw0, semw1)

        def process(src, tab, out, total, K):
            n_chunks = (total + _C - 1) // _C
            my_n = (n_chunks - wid + NW - 1) // NW
            off = field * K
            kf = float(K)

            def chunk_base(t):
                return jnp.minimum((wid + t * NW) * _C, total - _C) * _NF

            def fire_load(t, p):
                pltpu.async_copy(
                    src.at[pl.ds(chunk_base(t), _V)], attr.at[p], sema[p])

            def wait_load(p):
                pltpu.make_async_copy(
                    src.at[pl.ds(0, _V)], attr.at[p], sema[p]).wait()

            def drain_writes(p):
                pltpu.make_async_copy(
                    tab.at[pl.ds(0, _V)], rows.at[p], semw[p]).wait()

            def do_chunk(t, p):
                base = chunk_base(t)

                @pl.when(t + 1 < my_n)
                def _():
                    fire_load(t + 1, 1 - p)

                @pl.when(t >= 2)
                def _():
                    drain_writes(p)

                wait_load(p)
                gcopies = []
                for g in range(_G):
                    def jbody(j, c, g=g):
                        col = attr[p, pl.ds(g * 128 + j * 16, 16)]
                        v = (col * kf).astype(jnp.int32) + 1 + off
                        v = jnp.minimum(jnp.maximum(v, off), off + (K - 1))
                        idx[p, g, pl.ds(j * 16, 16)] = v
                        return c
                    lax.fori_loop(0, 8, jbody, 0)
                    gcopies.append(pltpu.async_copy(
                        tab.at[idx.at[p, g]],
                        rows.at[p, pl.ds(g * 128, 128)],
                        semg,
                    ))
                for cp in gcopies:
                    cp.wait()
                pltpu.async_copy(
                    rows.at[p], out.at[pl.ds(base, _V)], semw[p])

            @pl.when(my_n >= 1)
            def _():
                fire_load(0, 0)

            def pair(tt, c):
                t0 = 2 * tt
                do_chunk(t0, 0)

                @pl.when(t0 + 1 < my_n)
                def _():
                    do_chunk(t0 + 1, 1)
                return c

            lax.fori_loop(0, (my_n + 1) // 2, pair, 0)

            for p in range(2):
                outstanding = ((my_n >= 1) & ((my_n - 1) % 2 == p)) | (
                    (my_n >= 2) & (my_n % 2 == p))

                @pl.when(outstanding)
                def _(p=p):
                    drain_writes(p)

        process(xa, ntab, out_x, N, node_k)
        process(ea, etab, out_e, E, edge_k)

    return tok


@jax.jit
def kernel(x, edge_index, edge_attr, node_tables, edge_tables):
    del edge_index
    N = x.shape[0]
    E = edge_attr.shape[0]
    node_k = node_tables.shape[1]
    edge_k = edge_tables.shape[1]
    tok = _build(N, E, node_k, edge_k)
    out_x, out_e = tok(
        x.reshape(-1),
        edge_attr.reshape(-1),
        node_tables.reshape(-1, _SUB),
        edge_tables.reshape(-1, _SUB),
    )
    return (out_x.reshape(N, _NF * _SUB), out_e.reshape(E, _NF * _SUB))


# speedup vs baseline: 15.0315x; 1.1792x over previous
import functools

import jax
import jax.numpy as jnp
from jax import lax
from jax.experimental import pallas as pl
from jax.experimental.pallas import tpu as pltpu
from jax.experimental.pallas import tpu_sc as plsc

_NF = 8
_SUB = 16
_C = 256
_V = _C * _NF
_G = _V // 128
_BPC = _C // 128


@functools.lru_cache(maxsize=None)
def _build(N, E, node_k, edge_k):
    assert E % _C == 0
    NB = E // 128
    info = plsc.get_sparse_core_info()
    NC, NS = info.num_cores, info.num_subcores
    NW = NC * NS
    mesh = plsc.VectorSubcoreMesh(core_axis_name="c", subcore_axis_name="s")

    @functools.partial(
        pl.kernel,
        mesh=mesh,
        compiler_params=pltpu.CompilerParams(use_tc_tiling_on_sc=False),
        out_type=(
            jax.ShapeDtypeStruct((N * _NF, _SUB), jnp.float32),
            jax.ShapeDtypeStruct((E, _NF * _SUB), jnp.float32),
        ),
        scratch_types=[
            pltpu.VMEM((2, _V), jnp.float32),
            pltpu.VMEM((2, _BPC, _NF, 128), jnp.float32),
            pltpu.VMEM((2, _G, 128), jnp.int32),
            pltpu.VMEM((2, _V, _SUB), jnp.float32),
            pltpu.SemaphoreType.DMA,
            pltpu.SemaphoreType.DMA,
            pltpu.SemaphoreType.DMA,
            pltpu.SemaphoreType.DMA,
            pltpu.SemaphoreType.DMA,
        ],
    )
    def tok(xa, ea, ntab, etab, out_x, out_e,
            attr_f, attr_b, idx, rows, sema0, sema1, semg, semw0, semw1):
        wid = lax.axis_index("s") * NC + lax.axis_index("c")
        iota = lax.broadcasted_iota(jnp.int32, (16,), 0)
        sema = (sema0, sema1)
        semw = (semw0, semw1)

        def pipeline(my_n, fire_load, do_chunk, drain_writes):
            @pl.when(my_n >= 1)
            def _():
                fire_load(0, 0)

            def pair(tt, c):
                t0 = 2 * tt

                def full_chunk(t, p):
                    @pl.when(t + 1 < my_n)
                    def _():
                        fire_load(t + 1, 1 - p)

                    @pl.when(t >= 2)
                    def _():
                        drain_writes(p)

                    do_chunk(t, p)

                full_chunk(t0, 0)

                @pl.when(t0 + 1 < my_n)
                def _():
                    full_chunk(t0 + 1, 1)
                return c

            lax.fori_loop(0, (my_n + 1) // 2, pair, 0)

            for p in range(2):
                outstanding = ((my_n >= 1) & ((my_n - 1) % 2 == p)) | (
                    (my_n >= 2) & (my_n % 2 == p))

                @pl.when(outstanding)
                def _(p=p):
                    drain_writes(p)

        def gather_groups(p, tab, compute_group):
            gcopies = []
            for g in range(_G):
                compute_group(p, g)
                gcopies.append(pltpu.async_copy(
                    tab.at[idx.at[p, g]],
                    rows.at[p, pl.ds(g * 128, 128)],
                    semg,
                ))
            for cp in gcopies:
                cp.wait()

        def make_drain(tab):
            def drain_writes(p):
                pltpu.make_async_copy(
                    tab.at[pl.ds(0, _V)], rows.at[p], semw[p]).wait()
            return drain_writes

        def node_phase():
            K = node_k
            n_chunks = (N + _C - 1) // _C
            my_n = (n_chunks - wid + NW - 1) // NW
            off = jnp.bitwise_and(iota, _NF - 1) * K

            def chunk_base(t):
                return jnp.minimum((wid + t * NW) * _C, N - _C) * _NF

            def fire_load(t, p):
                pltpu.async_copy(
                    xa.at[pl.ds(chunk_base(t), _V)], attr_f.at[p], sema[p])

            def compute_group(p, g):
                def jbody(j, c):
                    col = attr_f[p, pl.ds(g * 128 + j * 16, 16)]
                    v = (col * float(K)).astype(jnp.int32) + 1 + off
                    v = jnp.minimum(jnp.maximum(v, off), off + (K - 1))
                    idx[p, g, pl.ds(j * 16, 16)] = v
                    return c
                lax.fori_loop(0, 8, jbody, 0)

            def do_chunk(t, p):
                pltpu.make_async_copy(
                    xa.at[pl.ds(0, _V)], attr_f.at[p], sema[p]).wait()
                gather_groups(p, ntab, compute_group)
                pltpu.async_copy(
                    rows.at[p], out_x.at[pl.ds(chunk_base(t), _V)], semw[p])

            pipeline(my_n, fire_load, do_chunk, make_drain(ntab))

        def edge_phase():
            K = edge_k
            n_chunks = NB // _BPC
            my_n = (n_chunks - wid + NW - 1) // NW

            def chunk_blk(t):
                return (wid + t * NW) * _BPC

            def fire_load(t, p):
                pltpu.async_copy(
                    ea.at[pl.ds(chunk_blk(t), _BPC)], attr_b.at[p], sema[p])

            def compute_group(p, g):
                bl, f = divmod(g, _NF)
                lo = jnp.int32(f * K)

                def jbody(j, c):
                    col = attr_b[p, bl, f, pl.ds(j * 16, 16)]
                    v = (col * float(K)).astype(jnp.int32) + (f * K + 1)
                    v = jnp.minimum(jnp.maximum(v, lo), lo + (K - 1))
                    idx[p, g, pl.ds(j * 16, 16)] = v
                    return c
                lax.fori_loop(0, 8, jbody, 0)

            def do_chunk(t, p):
                pltpu.make_async_copy(
                    ea.at[pl.ds(0, _BPC)], attr_b.at[p], sema[p]).wait()
                gather_groups(p, etab, compute_group)
                row0 = chunk_blk(t) * 128
                for g in range(_G):
                    bl, f = divmod(g, _NF)
                    pltpu.async_copy(
                        rows.at[p, pl.ds(g * 128, 128)],
                        out_e.at[pl.ds(row0 + bl * 128, 128),
                                 pl.ds(f * _SUB, _SUB)],
                        semw[p],
                    )

            pipeline(my_n, fire_load, do_chunk, make_drain(etab))

        node_phase()
        edge_phase()

    return tok


@jax.jit
def kernel(x, edge_index, edge_attr, node_tables, edge_tables):
    del edge_index
    N = x.shape[0]
    E = edge_attr.shape[0]
    node_k = node_tables.shape[1]
    edge_k = edge_tables.shape[1]
    tok = _build(N, E, node_k, edge_k)
    ea_blocked = edge_attr.reshape(E // 128, 128, _NF).transpose(0, 2, 1)
    out_x, out_e = tok(
        x.reshape(-1),
        ea_blocked,
        node_tables.reshape(-1, _SUB),
        edge_tables.reshape(-1, _SUB),
    )
    return (out_x.reshape(N, _NF * _SUB), out_e)


# speedup vs baseline: 15.5312x; 1.0332x over previous
import functools

import jax
import jax.numpy as jnp
from jax import lax
from jax.experimental import pallas as pl
from jax.experimental.pallas import tpu as pltpu
from jax.experimental.pallas import tpu_sc as plsc

_NF = 8
_SUB = 16
_C = 256
_V = _C * _NF
_G = _V // 128
_BPC = _C // 128


@functools.lru_cache(maxsize=None)
def _build(N, E, node_k, edge_k):
    assert E % _C == 0
    NB = E // 128
    info = plsc.get_sparse_core_info()
    NC, NS = info.num_cores, info.num_subcores
    NW = NC * NS
    mesh = plsc.VectorSubcoreMesh(core_axis_name="c", subcore_axis_name="s")

    @functools.partial(
        pl.kernel,
        mesh=mesh,
        compiler_params=pltpu.CompilerParams(use_tc_tiling_on_sc=False),
        out_type=(
            jax.ShapeDtypeStruct((N * _NF, _SUB), jnp.float32),
            jax.ShapeDtypeStruct((E, _NF * _SUB), jnp.float32),
        ),
        scratch_types=[
            pltpu.VMEM((2, _V), jnp.float32),
            pltpu.VMEM((2, _BPC, _NF, 128), jnp.float32),
            pltpu.VMEM((2, _G, 128), jnp.int32),
            pltpu.VMEM((2, _V, _SUB), jnp.float32),
            pltpu.SemaphoreType.DMA,
            pltpu.SemaphoreType.DMA,
            pltpu.SemaphoreType.DMA,
            pltpu.SemaphoreType.DMA,
            pltpu.SemaphoreType.DMA,
        ],
    )
    def tok(xa, ea, ntab, etab, out_x, out_e,
            attr_f, attr_b, idx, rows, sema0, sema1, semg, semw0, semw1):
        wid = lax.axis_index("s") * NC + lax.axis_index("c")
        iota = lax.broadcasted_iota(jnp.int32, (16,), 0)
        sema = (sema0, sema1)
        semw = (semw0, semw1)

        def pipeline(my_n, fire_load, do_chunk, drain_writes):
            @pl.when(my_n >= 1)
            def _():
                fire_load(0, 0)

            def pair(tt, c):
                t0 = 2 * tt

                def full_chunk(t, p):
                    @pl.when(t + 1 < my_n)
                    def _():
                        fire_load(t + 1, 1 - p)

                    @pl.when(t >= 2)
                    def _():
                        drain_writes(p)

                    do_chunk(t, p)

                full_chunk(t0, 0)

                @pl.when(t0 + 1 < my_n)
                def _():
                    full_chunk(t0 + 1, 1)
                return c

            lax.fori_loop(0, (my_n + 1) // 2, pair, 0)

            for p in range(2):
                outstanding = ((my_n >= 1) & ((my_n - 1) % 2 == p)) | (
                    (my_n >= 2) & (my_n % 2 == p))

                @pl.when(outstanding)
                def _(p=p):
                    drain_writes(p)

        def gather_groups(p, tab, compute_group, write_group):
            gcopies = []
            for g in range(_G):
                compute_group(p, g)
                gcopies.append(pltpu.async_copy(
                    tab.at[idx.at[p, g]],
                    rows.at[p, pl.ds(g * 128, 128)],
                    semg,
                ))
            for g, cp in enumerate(gcopies):
                cp.wait()
                write_group(p, g)

        def make_drain(tab):
            def drain_writes(p):
                pltpu.make_async_copy(
                    tab.at[pl.ds(0, _V)], rows.at[p], semw[p]).wait()
            return drain_writes

        def node_phase():
            K = node_k
            n_chunks = (N + _C - 1) // _C
            my_n = (n_chunks - wid + NW - 1) // NW
            off = jnp.bitwise_and(iota, _NF - 1) * K

            def chunk_base(t):
                return jnp.minimum((wid + t * NW) * _C, N - _C) * _NF

            def fire_load(t, p):
                pltpu.async_copy(
                    xa.at[pl.ds(chunk_base(t), _V)], attr_f.at[p], sema[p])

            def compute_group(p, g):
                def jbody(j, c):
                    col = attr_f[p, pl.ds(g * 128 + j * 16, 16)]
                    v = (col * float(K)).astype(jnp.int32) + 1 + off
                    v = jnp.minimum(jnp.maximum(v, off), off + (K - 1))
                    idx[p, g, pl.ds(j * 16, 16)] = v
                    return c
                lax.fori_loop(0, 8, jbody, 0)

            def do_chunk(t, p):
                pltpu.make_async_copy(
                    xa.at[pl.ds(0, _V)], attr_f.at[p], sema[p]).wait()
                base = chunk_base(t)

                def write_group(p, g):
                    pltpu.async_copy(
                        rows.at[p, pl.ds(g * 128, 128)],
                        out_x.at[pl.ds(base + g * 128, 128)],
                        semw[p],
                    )

                gather_groups(p, ntab, compute_group, write_group)

            pipeline(my_n, fire_load, do_chunk, make_drain(ntab))

        def edge_phase():
            K = edge_k
            n_chunks = NB // _BPC
            my_n = (n_chunks - wid + NW - 1) // NW

            def chunk_blk(t):
                return (wid + t * NW) * _BPC

            def fire_load(t, p):
                pltpu.async_copy(
                    ea.at[pl.ds(chunk_blk(t), _BPC)], attr_b.at[p], sema[p])

            def compute_group(p, g):
                bl, f = divmod(g, _NF)
                lo = jnp.int32(f * K)

                def jbody(j, c):
                    col = attr_b[p, bl, f, pl.ds(j * 16, 16)]
                    v = (col * float(K)).astype(jnp.int32) + (f * K + 1)
                    v = jnp.minimum(jnp.maximum(v, lo), lo + (K - 1))
                    idx[p, g, pl.ds(j * 16, 16)] = v
                    return c
                lax.fori_loop(0, 8, jbody, 0)

            def do_chunk(t, p):
                pltpu.make_async_copy(
                    ea.at[pl.ds(0, _BPC)], attr_b.at[p], sema[p]).wait()
                row0 = chunk_blk(t) * 128

                def write_group(p, g):
                    bl, f = divmod(g, _NF)
                    pltpu.async_copy(
                        rows.at[p, pl.ds(g * 128, 128)],
                        out_e.at[pl.ds(row0 + bl * 128, 128),
                                 pl.ds(f * _SUB, _SUB)],
                        semw[p],
                    )

                gather_groups(p, etab, compute_group, write_group)

            pipeline(my_n, fire_load, do_chunk, make_drain(etab))

        node_phase()
        edge_phase()

    return tok


@jax.jit
def kernel(x, edge_index, edge_attr, node_tables, edge_tables):
    del edge_index
    N = x.shape[0]
    E = edge_attr.shape[0]
    node_k = node_tables.shape[1]
    edge_k = edge_tables.shape[1]
    tok = _build(N, E, node_k, edge_k)
    ea_blocked = edge_attr.reshape(E // 128, 128, _NF).transpose(0, 2, 1)
    out_x, out_e = tok(
        x.reshape(-1),
        ea_blocked,
        node_tables.reshape(-1, _SUB),
        edge_tables.reshape(-1, _SUB),
    )
    return (out_x.reshape(N, _NF * _SUB), out_e)


# speedup vs baseline: 17.8422x; 1.1488x over previous
import functools

import jax
import jax.numpy as jnp
from jax import lax
from jax.experimental import pallas as pl
from jax.experimental.pallas import tpu as pltpu
from jax.experimental.pallas import tpu_sc as plsc

_NF = 8
_SUB = 16
_C = 256
_V = _C * _NF
_G = _V // 128
_BPC = _C // 128


def _worker_id(NC):
    return lax.axis_index("s") * NC + lax.axis_index("c")


def _pipeline(my_n, fire_load, do_chunk, drain_writes):
    @pl.when(my_n >= 1)
    def _():
        fire_load(0, 0)

    def pair(tt, c):
        t0 = 2 * tt

        def full_chunk(t, p):
            @pl.when(t + 1 < my_n)
            def _():
                fire_load(t + 1, 1 - p)

            @pl.when(t >= 2)
            def _():
                drain_writes(p)

            do_chunk(t, p)

        full_chunk(t0, 0)

        @pl.when(t0 + 1 < my_n)
        def _():
            full_chunk(t0 + 1, 1)
        return c

    lax.fori_loop(0, (my_n + 1) // 2, pair, 0)

    for p in range(2):
        outstanding = ((my_n >= 1) & ((my_n - 1) % 2 == p)) | (
            (my_n >= 2) & (my_n % 2 == p))

        @pl.when(outstanding)
        def _(p=p):
            drain_writes(p)


def _gather_groups(p, tab, idx, rows, semg, compute_group, write_group):
    gcopies = []
    for g in range(_G):
        compute_group(p, g)
        gcopies.append(pltpu.async_copy(
            tab.at[idx.at[p, g]],
            rows.at[p, pl.ds(g * 128, 128)],
            semg,
        ))
    for g, cp in enumerate(gcopies):
        cp.wait()
        write_group(p, g)


_SCRATCH_COMMON = [
    pltpu.VMEM((2, _G, 128), jnp.int32),
    pltpu.VMEM((2, _V, _SUB), jnp.float32),
    pltpu.SemaphoreType.DMA,
    pltpu.SemaphoreType.DMA,
    pltpu.SemaphoreType.DMA,
    pltpu.SemaphoreType.DMA,
    pltpu.SemaphoreType.DMA,
]


@functools.lru_cache(maxsize=None)
def _build_node(N, K):
    info = plsc.get_sparse_core_info()
    NC, NW = info.num_cores, info.num_cores * info.num_subcores
    mesh = plsc.VectorSubcoreMesh(core_axis_name="c", subcore_axis_name="s")

    @functools.partial(
        pl.kernel,
        mesh=mesh,
        compiler_params=pltpu.CompilerParams(use_tc_tiling_on_sc=False),
        out_type=jax.ShapeDtypeStruct((N * _NF, _SUB), jnp.float32),
        scratch_types=[pltpu.VMEM((2, _V), jnp.float32)] + _SCRATCH_COMMON,
    )
    def tok_node(xa, ntab, out_x,
                 attr_f, idx, rows, sema0, sema1, semg, semw0, semw1):
        wid = _worker_id(NC)
        iota = lax.broadcasted_iota(jnp.int32, (16,), 0)
        off = jnp.bitwise_and(iota, _NF - 1) * K
        sema = (sema0, sema1)
        semw = (semw0, semw1)

        n_chunks = (N + _C - 1) // _C
        my_n = (n_chunks - wid + NW - 1) // NW

        def chunk_base(t):
            return jnp.minimum((wid + t * NW) * _C, N - _C) * _NF

        def fire_load(t, p):
            pltpu.async_copy(
                xa.at[pl.ds(chunk_base(t), _V)], attr_f.at[p], sema[p])

        def drain_writes(p):
            pltpu.make_async_copy(
                ntab.at[pl.ds(0, _V)], rows.at[p], semw[p]).wait()

        def compute_group(p, g):
            def jbody(j, c):
                col = attr_f[p, pl.ds(g * 128 + j * 16, 16)]
                v = (col * float(K)).astype(jnp.int32) + 1 + off
                v = jnp.minimum(jnp.maximum(v, off), off + (K - 1))
                idx[p, g, pl.ds(j * 16, 16)] = v
                return c
            lax.fori_loop(0, 8, jbody, 0)

        def do_chunk(t, p):
            pltpu.make_async_copy(
                xa.at[pl.ds(0, _V)], attr_f.at[p], sema[p]).wait()
            base = chunk_base(t)

            def write_group(p, g):
                pltpu.async_copy(
                    rows.at[p, pl.ds(g * 128, 128)],
                    out_x.at[pl.ds(base + g * 128, 128)],
                    semw[p],
                )

            _gather_groups(p, ntab, idx, rows, semg,
                           compute_group, write_group)

        _pipeline(my_n, fire_load, do_chunk, drain_writes)

    return tok_node


@functools.lru_cache(maxsize=None)
def _build_edge(E, K):
    assert E % _C == 0
    NB = E // 128
    info = plsc.get_sparse_core_info()
    NC, NW = info.num_cores, info.num_cores * info.num_subcores
    mesh = plsc.VectorSubcoreMesh(core_axis_name="c", subcore_axis_name="s")

    @functools.partial(
        pl.kernel,
        mesh=mesh,
        compiler_params=pltpu.CompilerParams(use_tc_tiling_on_sc=False),
        out_type=jax.ShapeDtypeStruct((E, _NF * _SUB), jnp.float32),
        scratch_types=(
            [pltpu.VMEM((2, _BPC, _NF, 128), jnp.float32)] + _SCRATCH_COMMON),
    )
    def tok_edge(ea, etab, out_e,
                 attr_b, idx, rows, sema0, sema1, semg, semw0, semw1):
        wid = _worker_id(NC)
        sema = (sema0, sema1)
        semw = (semw0, semw1)

        n_chunks = NB // _BPC
        my_n = (n_chunks - wid + NW - 1) // NW

        def chunk_blk(t):
            return (wid + t * NW) * _BPC

        def fire_load(t, p):
            pltpu.async_copy(
                ea.at[pl.ds(chunk_blk(t), _BPC)], attr_b.at[p], sema[p])

        def drain_writes(p):
            pltpu.make_async_copy(
                etab.at[pl.ds(0, _V)], rows.at[p], semw[p]).wait()

        def compute_group(p, g):
            bl, f = divmod(g, _NF)
            lo = jnp.int32(f * K)

            def jbody(j, c):
                col = attr_b[p, bl, f, pl.ds(j * 16, 16)]
                v = (col * float(K)).astype(jnp.int32) + (f * K + 1)
                v = jnp.minimum(jnp.maximum(v, lo), lo + (K - 1))
                idx[p, g, pl.ds(j * 16, 16)] = v
                return c
            lax.fori_loop(0, 8, jbody, 0)

        def do_chunk(t, p):
            pltpu.make_async_copy(
                ea.at[pl.ds(0, _BPC)], attr_b.at[p], sema[p]).wait()
            row0 = chunk_blk(t) * 128

            def write_group(p, g):
                bl, f = divmod(g, _NF)
                pltpu.async_copy(
                    rows.at[p, pl.ds(g * 128, 128)],
                    out_e.at[pl.ds(row0 + bl * 128, 128),
                             pl.ds(f * _SUB, _SUB)],
                    semw[p],
                )

            _gather_groups(p, etab, idx, rows, semg,
                           compute_group, write_group)

        _pipeline(my_n, fire_load, do_chunk, drain_writes)

    return tok_edge


@jax.jit
def kernel(x, edge_index, edge_attr, node_tables, edge_tables):
    del edge_index
    N = x.shape[0]
    E = edge_attr.shape[0]
    node_k = node_tables.shape[1]
    edge_k = edge_tables.shape[1]
    ea_blocked = edge_attr.reshape(E // 128, 128, _NF).transpose(0, 2, 1)
    out_e = _build_edge(E, edge_k)(ea_blocked, edge_tables.reshape(-1, _SUB))
    out_x = _build_node(N, node_k)(x.reshape(-1), node_tables.reshape(-1, _SUB))
    return (out_x.reshape(N, _NF * _SUB), out_e)
